# baseline (device time: 55478 ns/iter reference)
import jax
import jax.numpy as jnp
from jax import lax
from jax.experimental import pallas as pl
from jax.experimental.pallas import tpu as pltpu

N_DEV = 16
B, Sq, Skv, Hq, Dh = 2, 512, 512, 128, 64
H_LOC = Hq // N_DEV
DM = 768
DH_LOC = H_LOC * Dh
ROWS = B * Sq
QR = ROWS // 4
CH = ROWS // N_DEV

_MESH = pl.DeviceIdType.MESH
_QXOR_OF_PXOR = [0, 2, 3, 1]

_EARLY = [(px, zx) for px in (1, 2) for zx in range(4)]
_LATE = [(px, zx) for px in (0, 3) for zx in range(4) if (px, zx) != (0, 0)]
_ROLES = _EARLY + _LATE


def _fused_body(x_ref, wq_ref, kt_ref, vt_ref, wo_ref, out_ref,
                q_buf, ctx_buf, acc, ps, pr,
                rs_s, rs_r, ag_s, ag_r):
    bf = jnp.bfloat16
    my = lax.axis_index("i")
    z = my // 4
    p = my % 4
    xb = jnp.where((p == 1) | (p == 2), 1, 0).astype(jnp.int32)
    yb = (p // 2).astype(jnp.int32)
    q = xb * 2 + yb
    fin = pl.multiple_of(q * QR + z * CH, CH)

    def peer_of(role):
        px, zx = role
        return jnp.bitwise_xor(z, zx) * 4 + jnp.bitwise_xor(p, px)

    def peer_fin(role):
        px, zx = role
        pq = jnp.bitwise_xor(q, _QXOR_OF_PXOR[px])
        return pl.multiple_of(pq * QR + jnp.bitwise_xor(z, zx) * CH, CH)

    peers = [peer_of(r) for r in _ROLES]

    barrier_sem = pltpu.get_barrier_semaphore()
    for nbr in peers:
        pl.semaphore_signal(
            barrier_sem, inc=1, device_id=(nbr,), device_id_type=_MESH,
        )

    q_buf[...] = (
        jnp.dot(x_ref[...], wq_ref[...], preferred_element_type=jnp.float32)
        * 0.125
    ).astype(bf)

    qi = lax.broadcasted_iota(jnp.int32, (Sq, Skv), 0)
    ki = lax.broadcasted_iota(jnp.int32, (Sq, Skv), 1)
    mask = (jnp.abs(qi - ki) <= 128) | (ki < 32) | (qi < 32)

    def attn_batch(bs):
        for h in range(H_LOC):
            bh = bs * H_LOC + h
            qh = q_buf[bs * Sq:(bs + 1) * Sq, h * Dh:(h + 1) * Dh]
            k = kt_ref[bh]
            v = vt_ref[bh]
            scores = lax.dot_general(
                qh, k, (((1,), (1,)), ((), ())),
                preferred_element_type=jnp.float32,
            )
            scores = jnp.where(mask, scores, -1e9)
            m = jnp.max(scores, axis=1, keepdims=True)
            e = jnp.exp(scores - m)
            w = (e / jnp.sum(e, axis=1, keepdims=True)).astype(bf)
            ctx = lax.dot_general(
                w, v, (((1,), (0,)), ((), ())),
                preferred_element_type=jnp.float32,
            )
            ctx_buf[bs * Sq:(bs + 1) * Sq, h * Dh:(h + 1) * Dh] = ctx.astype(bf)

    def partial_batch(bs):
        return jnp.dot(
            ctx_buf[bs * Sq:(bs + 1) * Sq, :], wo_ref[...],
            preferred_element_type=jnp.float32,
        )

    def rs_desc(slot):
        return pltpu.make_async_remote_copy(
            src_ref=ps.at[slot],
            dst_ref=pr.at[slot],
            send_sem=rs_s.at[slot],
            recv_sem=rs_r.at[slot],
            device_id=(peers[slot],),
            device_id_type=_MESH,
        )

    def stage_and_start(slot):
        ps[slot, :, :] = acc[pl.ds(peer_fin(_ROLES[slot]), CH), :].astype(bf)
        d = rs_desc(slot)
        d.start()
        return d

    @pl.when(xb == 0)
    def _():
        attn_batch(1)
        acc[Sq:2 * Sq, :] = partial_batch(1)

    @pl.when(xb == 1)
    def _():
        attn_batch(0)
        acc[0:Sq, :] = partial_batch(0)

    pl.semaphore_wait(barrier_sem, 15)
    d_rs = {}
    for slot in range(len(_EARLY)):
        d_rs[slot] = stage_and_start(slot)

    @pl.when(xb == 0)
    def _():
        attn_batch(0)
        acc[0:Sq, :] = partial_batch(0)

    @pl.when(xb == 1)
    def _():
        attn_batch(1)
        acc[Sq:2 * Sq, :] = partial_batch(1)

    for slot in range(len(_EARLY), len(_ROLES)):
        d_rs[slot] = stage_and_start(slot)

    for slot in range(len(_ROLES)):
        d_rs[slot].wait()
        acc[pl.ds(fin, CH), :] = (
            acc[pl.ds(fin, CH), :] + pr[slot].astype(jnp.float32)
        )
    out_ref[pl.ds(fin, CH), :] = acc[pl.ds(fin, CH), :].astype(bf)

    d_ag = []
    for slot in range(len(_ROLES)):
        d = pltpu.make_async_remote_copy(
            src_ref=out_ref.at[pl.ds(fin, CH), :],
            dst_ref=out_ref.at[pl.ds(fin, CH), :],
            send_sem=ag_s.at[slot],
            recv_sem=ag_r.at[slot],
            device_id=(peers[slot],),
            device_id_type=_MESH,
        )
        d.start()
        d_ag.append(d)
    for d in d_ag:
        d.wait()


def kernel(x, Wq, K_ext, V_ext, Wo):
    my = lax.axis_index("i")
    bf = jnp.bfloat16

    Wq_loc = lax.dynamic_slice(Wq, (0, my * DH_LOC), (DM, DH_LOC)).astype(bf)
    Wo_loc = lax.dynamic_slice(Wo, (my * DH_LOC, 0), (DH_LOC, DM)).astype(bf)
    x_bf = x.reshape(ROWS, DM).astype(bf)
    K_t = K_ext.transpose(0, 2, 1, 3).reshape(B * H_LOC, Skv, Dh).astype(bf)
    V_t = V_ext.transpose(0, 2, 1, 3).reshape(B * H_LOC, Skv, Dh).astype(bf)

    vmem = pl.BlockSpec(memory_space=pltpu.VMEM)
    out = pl.pallas_call(
        _fused_body,
        out_shape=jax.ShapeDtypeStruct((ROWS, DM), bf),
        in_specs=[vmem] * 5,
        out_specs=vmem,
        scratch_shapes=[
            pltpu.VMEM((ROWS, DH_LOC), bf),
            pltpu.VMEM((ROWS, DH_LOC), bf),
            pltpu.VMEM((ROWS, DM), jnp.float32),
            pltpu.VMEM((15, CH, DM), bf),
            pltpu.VMEM((15, CH, DM), bf),
            pltpu.SemaphoreType.DMA((15,)),
            pltpu.SemaphoreType.DMA((15,)),
            pltpu.SemaphoreType.DMA((15,)),
            pltpu.SemaphoreType.DMA((15,)),
        ],
        compiler_params=pltpu.CompilerParams(collective_id=0),
    )(x_bf, Wq_loc, K_t, V_t, Wo_loc)
    return out.reshape(B, Sq, DM)
